# baseline (speedup 1.0000x reference)
import jax
import jax.numpy as jnp
from jax.experimental import pallas as pl
from jax.experimental.pallas import tpu as pltpu

EPS = 1e-05
_BLOCK_COLS = 4096


def _pdf_block(w_ref, o_ref):
    w = w_ref[...]
    s = jnp.sum(w, axis=0, keepdims=True)
    pad = jnp.maximum(EPS - s, 0.0)
    inv = 1.0 / (s + pad)
    o_ref[...] = (w + pad * (1.0 / w.shape[0])) * inv


def kernel(weights, stratified):
    n, d = weights.shape
    wt = weights.T
    out_t = pl.pallas_call(
        _pdf_block,
        grid=(n // _BLOCK_COLS,),
        in_specs=[pl.BlockSpec((d, _BLOCK_COLS), lambda i: (0, i))],
        out_specs=pl.BlockSpec((d, _BLOCK_COLS), lambda i: (0, i)),
        out_shape=jax.ShapeDtypeStruct((d, n), weights.dtype),
        compiler_params=pltpu.CompilerParams(
            dimension_semantics=("parallel",),
        ),
    )(wt)
    return out_t.T


# speedup vs baseline: 1.3715x; 1.3715x over previous
import jax
import jax.numpy as jnp
from jax.experimental import pallas as pl
from jax.experimental.pallas import tpu as pltpu

EPS = 1e-05
_BLOCK_COLS = 16384


def _pdf_block(w_ref, o_ref):
    w = w_ref[...]
    s = jnp.sum(w, axis=0, keepdims=True)
    pad = jnp.maximum(EPS - s, 0.0)
    inv = 1.0 / (s + pad)
    o_ref[...] = (w + pad * (1.0 / w.shape[0])) * inv


def kernel(weights, stratified):
    n, d = weights.shape
    wt = weights.T
    out_t = pl.pallas_call(
        _pdf_block,
        grid=(n // _BLOCK_COLS,),
        in_specs=[pl.BlockSpec((d, _BLOCK_COLS), lambda i: (0, i))],
        out_specs=pl.BlockSpec((d, _BLOCK_COLS), lambda i: (0, i)),
        out_shape=jax.ShapeDtypeStruct((d, n), weights.dtype),
        compiler_params=pltpu.CompilerParams(
            dimension_semantics=("parallel",),
        ),
    )(wt)
    return out_t.T


# speedup vs baseline: 1.6180x; 1.1797x over previous
import jax
import jax.numpy as jnp
from jax.experimental import pallas as pl
from jax.experimental.pallas import tpu as pltpu

EPS = 1e-05
_BLOCK_COLS = 32768


def _pdf_block(w_ref, o_ref):
    w = w_ref[...]
    s = jnp.sum(w, axis=0, keepdims=True)
    pad = jnp.maximum(EPS - s, 0.0)
    inv = 1.0 / (s + pad)
    o_ref[...] = (w + pad * (1.0 / w.shape[0])) * inv


def kernel(weights, stratified):
    n, d = weights.shape
    wt = weights.T
    out_t = pl.pallas_call(
        _pdf_block,
        grid=(n // _BLOCK_COLS,),
        in_specs=[pl.BlockSpec((d, _BLOCK_COLS), lambda i: (0, i))],
        out_specs=pl.BlockSpec((d, _BLOCK_COLS), lambda i: (0, i)),
        out_shape=jax.ShapeDtypeStruct((d, n), weights.dtype),
        compiler_params=pltpu.CompilerParams(
            dimension_semantics=("parallel",),
        ),
    )(wt)
    return out_t.T
